# baseline (speedup 1.0000x reference)
import functools

import jax
import jax.numpy as jnp
from jax.experimental import pallas as pl
from jax.experimental.pallas import tpu as pltpu

_F32 = jnp.float32


def _body(pts_ref, rot_ref, tr_ref, feat_ref, mask_ref, wp_ref, wc_ref,
          bc_ref, bp_ref, w1_ref, b1_ref, w2_ref, b2_ref, wout_ref, bout_ref,
          ptm_ref, imp_ref, a_s, g_s, bias2_s, *, M, H):
    i = pl.program_id(1)

    @pl.when(i == 0)
    def _prep():
        q = rot_ref[0]
        norm = jnp.sqrt(jnp.sum(q * q, axis=1, keepdims=True))
        qn = q / jnp.maximum(norm, 1e-8)
        qw = qn[:, 0:1]
        qx = qn[:, 1:2]
        qy = qn[:, 2:3]
        qz = qn[:, 3:4]
        xx = qx * qx
        yy = qy * qy
        zz = qz * qz
        xy = qx * qy
        xz = qx * qz
        yz = qy * qz
        wx = qw * qx
        wy = qw * qy
        wz = qw * qz
        one = jnp.ones_like(qw)
        r00 = one - 2.0 * (yy + zz)
        r01 = 2.0 * (xy + wz)
        r02 = 2.0 * (xz - wy)
        r10 = 2.0 * (xy - wz)
        r11 = one - 2.0 * (xx + zz)
        r12 = 2.0 * (yz + wx)
        r20 = 2.0 * (xz + wy)
        r21 = 2.0 * (yz - wx)
        r22 = one - 2.0 * (xx + yy)
        t = tr_ref[0]
        tx = t[:, 0:1]
        ty = t[:, 1:2]
        tz = t[:, 2:3]
        c0 = -(r00 * tx + r01 * ty + r02 * tz)
        c1 = -(r10 * tx + r11 * ty + r12 * tz)
        c2 = -(r20 * tx + r21 * ty + r22 * tz)
        bias = (jnp.dot(feat_ref[0], wc_ref[...], preferred_element_type=_F32)
                + bc_ref[...] + bp_ref[...])
        wp = wp_ref[...]
        for m in range(M):
            s = slice(m, m + 1)
            row0 = jnp.concatenate([r00[s], r10[s], r20[s]], axis=1)
            row1 = jnp.concatenate([r01[s], r11[s], r21[s]], axis=1)
            row2 = jnp.concatenate([r02[s], r12[s], r22[s]], axis=1)
            crow = jnp.concatenate([c0[s], c1[s], c2[s]], axis=1)
            aaug = jnp.concatenate([row0, row1, row2, crow], axis=0)
            a_s[0:4, m * 3:(m + 1) * 3] = aaug
            gaug = jnp.dot(aaug, wp, preferred_element_type=_F32)
            g_s[0:3, m * H:(m + 1) * H] = gaug[0:3, :]
            bias2_s[s, :] = bias[s, :] + gaug[3:4, :]

    x = pts_ref[0]
    ptm_ref[0] = (jnp.dot(x, a_s[0:3, :], preferred_element_type=_F32)
                  + a_s[3:4, :])
    w1 = w1_ref[...]
    b1 = b1_ref[...]
    w2 = w2_ref[...]
    b2 = b2_ref[...]
    wout = wout_ref[...]
    bout = bout_ref[...]
    occ_cols = []
    for m in range(M):
        net = (jnp.dot(x, g_s[0:3, m * H:(m + 1) * H],
                       preferred_element_type=_F32) + bias2_s[m:m + 1, :])
        h = jnp.dot(jnp.maximum(net, 0.0), w1,
                    preferred_element_type=_F32) + b1
        h = jnp.dot(jnp.maximum(h, 0.0), w2,
                    preferred_element_type=_F32) + b2
        net = net + h
        occ_cols.append(
            jnp.sum(jnp.maximum(net, 0.0) * wout, axis=1, keepdims=True)
            + bout)
    occ = jnp.concatenate(occ_cols, axis=1)
    imp_ref[0] = jnp.where(mask_ref[0], jax.nn.sigmoid(10.0 * occ), 0.0)


def kernel(ray_points, translations, rotations, part_shape_features,
           points_mask, W_p, b_p, W_c, b_c, W1, b1, W2, b2, W_out, b_out):
    B, N, P, _ = ray_points.shape
    M = translations.shape[1]
    C = part_shape_features.shape[-1]
    H = W_p.shape[1]
    Q = N * P

    T = 128
    grid = (B, Q // T)
    ptm, imp = pl.pallas_call(
        functools.partial(_body, M=M, H=H),
        grid=grid,
        in_specs=[
            pl.BlockSpec((1, T, 3), lambda b, i: (b, i, 0)),
            pl.BlockSpec((1, M, 4), lambda b, i: (b, 0, 0)),
            pl.BlockSpec((1, M, 3), lambda b, i: (b, 0, 0)),
            pl.BlockSpec((1, M, C), lambda b, i: (b, 0, 0)),
            pl.BlockSpec((1, T, M), lambda b, i: (b, i, 0)),
            pl.BlockSpec((3, H), lambda b, i: (0, 0)),
            pl.BlockSpec((C, H), lambda b, i: (0, 0)),
            pl.BlockSpec((1, H), lambda b, i: (0, 0)),
            pl.BlockSpec((1, H), lambda b, i: (0, 0)),
            pl.BlockSpec((H, H), lambda b, i: (0, 0)),
            pl.BlockSpec((1, H), lambda b, i: (0, 0)),
            pl.BlockSpec((H, H), lambda b, i: (0, 0)),
            pl.BlockSpec((1, H), lambda b, i: (0, 0)),
            pl.BlockSpec((1, H), lambda b, i: (0, 0)),
            pl.BlockSpec((1, 1), lambda b, i: (0, 0)),
        ],
        out_specs=[
            pl.BlockSpec((1, T, M * 3), lambda b, i: (b, i, 0)),
            pl.BlockSpec((1, T, M), lambda b, i: (b, i, 0)),
        ],
        out_shape=(
            jax.ShapeDtypeStruct((B, Q, M * 3), _F32),
            jax.ShapeDtypeStruct((B, Q, M), _F32),
        ),
        scratch_shapes=[
            pltpu.VMEM((8, M * 3), _F32),
            pltpu.VMEM((8, M * H), _F32),
            pltpu.VMEM((M, H), _F32),
        ],
        compiler_params=pltpu.CompilerParams(
            dimension_semantics=("arbitrary", "arbitrary")),
    )(
        ray_points.reshape(B, Q, 3), rotations, translations,
        part_shape_features, points_mask.reshape(B, Q, M), W_p, W_c,
        b_c.reshape(1, H), b_p.reshape(1, H), W1, b1.reshape(1, H),
        W2, b2.reshape(1, H), W_out.reshape(1, H), b_out.reshape(1, 1),
    )

    implicit_field = imp.reshape(B, N, P, M)
    points_transformed = ptm.reshape(B, N, P, M * 3)
    return implicit_field, points_transformed


# speedup vs baseline: 1.5245x; 1.5245x over previous
import functools

import jax
import jax.numpy as jnp
from jax.experimental import pallas as pl
from jax.experimental.pallas import tpu as pltpu

_F32 = jnp.float32


def _body(pts_ref, rot_ref, tr_ref, feat_ref, mask_ref, wp_ref, wc_ref,
          bc_ref, bp_ref, w1_ref, b1_ref, w2_ref, b2_ref, wout_ref, bout_ref,
          ptm_ref, imp_ref, a_s, g_s, *, M, H):
    i = pl.program_id(1)

    @pl.when(i == 0)
    def _prep():
        q = rot_ref[0]
        norm = jnp.sqrt(jnp.sum(q * q, axis=1, keepdims=True))
        qn = q / jnp.maximum(norm, 1e-8)
        qw = qn[:, 0:1]
        qx = qn[:, 1:2]
        qy = qn[:, 2:3]
        qz = qn[:, 3:4]
        xx = qx * qx
        yy = qy * qy
        zz = qz * qz
        xy = qx * qy
        xz = qx * qz
        yz = qy * qz
        wx = qw * qx
        wy = qw * qy
        wz = qw * qz
        one = jnp.ones_like(qw)
        r00 = one - 2.0 * (yy + zz)
        r01 = 2.0 * (xy + wz)
        r02 = 2.0 * (xz - wy)
        r10 = 2.0 * (xy - wz)
        r11 = one - 2.0 * (xx + zz)
        r12 = 2.0 * (yz + wx)
        r20 = 2.0 * (xz + wy)
        r21 = 2.0 * (yz - wx)
        r22 = one - 2.0 * (xx + yy)
        t = tr_ref[0]
        tx = t[:, 0:1]
        ty = t[:, 1:2]
        tz = t[:, 2:3]
        c0 = -(r00 * tx + r01 * ty + r02 * tz)
        c1 = -(r10 * tx + r11 * ty + r12 * tz)
        c2 = -(r20 * tx + r21 * ty + r22 * tz)
        bias = (jnp.dot(feat_ref[0], wc_ref[...], preferred_element_type=_F32)
                + bc_ref[...] + bp_ref[...])
        wp = wp_ref[...]
        for m in range(M):
            s = slice(m, m + 1)
            row0 = jnp.concatenate([r00[s], r10[s], r20[s]], axis=1)
            row1 = jnp.concatenate([r01[s], r11[s], r21[s]], axis=1)
            row2 = jnp.concatenate([r02[s], r12[s], r22[s]], axis=1)
            crow = jnp.concatenate([c0[s], c1[s], c2[s]], axis=1)
            aaug = jnp.concatenate([row0, row1, row2, crow], axis=0)
            a_s[0:4, m * 3:(m + 1) * 3] = aaug
            gaug = jnp.dot(aaug, wp, preferred_element_type=_F32)
            g_s[0:4, m * H:(m + 1) * H] = jnp.concatenate(
                [gaug[0:3, :], bias[s, :] + gaug[3:4, :]], axis=0)

    x = pts_ref[0]
    x4 = jnp.concatenate([x, jnp.ones_like(x[:, 0:1])], axis=1)
    ptm_ref[0] = jnp.dot(x4, a_s[0:4, :], preferred_element_type=_F32)
    w1 = w1_ref[...]
    b1 = b1_ref[...]
    w2 = w2_ref[...]
    b2 = b2_ref[...]
    wout = wout_ref[...]
    bout = bout_ref[...]
    occ_cols = []
    for m in range(M):
        net = jnp.dot(x4, g_s[0:4, m * H:(m + 1) * H],
                      preferred_element_type=_F32)
        h = jnp.dot(jnp.maximum(net, 0.0), w1,
                    preferred_element_type=_F32) + b1
        h = jnp.dot(jnp.maximum(h, 0.0), w2,
                    preferred_element_type=_F32) + b2
        net = net + h
        occ_cols.append(
            jnp.sum(jnp.maximum(net, 0.0) * wout, axis=1, keepdims=True)
            + bout)
    occ = jnp.concatenate(occ_cols, axis=1)
    imp_ref[0] = jnp.where(mask_ref[0], jax.nn.sigmoid(10.0 * occ), 0.0)


def kernel(ray_points, translations, rotations, part_shape_features,
           points_mask, W_p, b_p, W_c, b_c, W1, b1, W2, b2, W_out, b_out):
    B, N, P, _ = ray_points.shape
    M = translations.shape[1]
    C = part_shape_features.shape[-1]
    H = W_p.shape[1]
    Q = N * P

    T = 256
    grid = (B, Q // T)
    ptm, imp = pl.pallas_call(
        functools.partial(_body, M=M, H=H),
        grid=grid,
        in_specs=[
            pl.BlockSpec((1, T, 3), lambda b, i: (b, i, 0)),
            pl.BlockSpec((1, M, 4), lambda b, i: (b, 0, 0)),
            pl.BlockSpec((1, M, 3), lambda b, i: (b, 0, 0)),
            pl.BlockSpec((1, M, C), lambda b, i: (b, 0, 0)),
            pl.BlockSpec((1, T, M), lambda b, i: (b, i, 0)),
            pl.BlockSpec((3, H), lambda b, i: (0, 0)),
            pl.BlockSpec((C, H), lambda b, i: (0, 0)),
            pl.BlockSpec((1, H), lambda b, i: (0, 0)),
            pl.BlockSpec((1, H), lambda b, i: (0, 0)),
            pl.BlockSpec((H, H), lambda b, i: (0, 0)),
            pl.BlockSpec((1, H), lambda b, i: (0, 0)),
            pl.BlockSpec((H, H), lambda b, i: (0, 0)),
            pl.BlockSpec((1, H), lambda b, i: (0, 0)),
            pl.BlockSpec((1, H), lambda b, i: (0, 0)),
            pl.BlockSpec((1, 1), lambda b, i: (0, 0)),
        ],
        out_specs=[
            pl.BlockSpec((1, T, M * 3), lambda b, i: (b, i, 0)),
            pl.BlockSpec((1, T, M), lambda b, i: (b, i, 0)),
        ],
        out_shape=(
            jax.ShapeDtypeStruct((B, Q, M * 3), _F32),
            jax.ShapeDtypeStruct((B, Q, M), _F32),
        ),
        scratch_shapes=[
            pltpu.VMEM((8, M * 3), _F32),
            pltpu.VMEM((8, M * H), _F32),
        ],
        compiler_params=pltpu.CompilerParams(
            dimension_semantics=("arbitrary", "arbitrary")),
    )(
        ray_points.reshape(B, Q, 3), rotations, translations,
        part_shape_features, points_mask.reshape(B, Q, M), W_p, W_c,
        b_c.reshape(1, H), b_p.reshape(1, H), W1, b1.reshape(1, H),
        W2, b2.reshape(1, H), W_out.reshape(1, H), b_out.reshape(1, 1),
    )

    implicit_field = imp.reshape(B, N, P, M)
    points_transformed = ptm.reshape(B, N, P, M * 3)
    return implicit_field, points_transformed
